# baseline (speedup 1.0000x reference)
import functools

import jax
import jax.numpy as jnp
from jax import lax
from jax.experimental import pallas as pl
from jax.experimental.pallas import tpu as pltpu
from jax.experimental.pallas import tpu_sc as plsc


_LANES = 16


def _sort_key(x):
    xz = x + 0.0
    b = lax.bitcast_convert_type(xz, jnp.int32)
    neg = jnp.bitwise_xor(jnp.bitwise_not(b), jnp.int32(-2147483648))
    return jnp.where(b >= 0, b, neg)


def _keys_kernel(s_ref, k_ref):
    k_ref[...] = _sort_key(s_ref[...])


def _count_kernel(kcol_ref, krow_ref, out_ref, acc_ref, *, bj, njt):
    jt = pl.program_id(2)
    m = krow_ref[0] > kcol_ref[0]
    mf = jnp.where(m, jnp.float32(1.0), jnp.float32(0.0))
    partial = jnp.dot(
        mf, jnp.ones((bj, 1), jnp.float32),
        preferred_element_type=jnp.float32,
    )

    @pl.when(jt == 0)
    def _():
        acc_ref[...] = partial

    @pl.when(jt != 0)
    def _():
        acc_ref[...] += partial

    @pl.when(jt == njt - 1)
    def _():
        out_ref[0] = acc_ref[...].astype(jnp.int32)


def _compute_ranks(scores):
    b, n = scores.shape
    keys = pl.pallas_call(
        _keys_kernel,
        out_shape=jax.ShapeDtypeStruct((b, n), jnp.int32),
    )(scores)
    keys_col = keys.reshape(b, n, 1)
    keys_row = keys.reshape(b, 1, n)

    bi, bj = 1024, 1024
    nit, njt = n // bi, n // bj

    ranks3d = pl.pallas_call(
        functools.partial(_count_kernel, bj=bj, njt=njt),
        grid=(b, nit, njt),
        in_specs=[
            pl.BlockSpec((1, bi, 1), lambda bb, it, jt: (bb, it, 0)),
            pl.BlockSpec((1, 1, bj), lambda bb, it, jt: (bb, 0, jt)),
        ],
        out_specs=pl.BlockSpec((1, bi, 1), lambda bb, it, jt: (bb, it, 0)),
        out_shape=jax.ShapeDtypeStruct((b, n, 1), jnp.int32),
        scratch_shapes=[pltpu.VMEM((bi, 1), jnp.float32)],
    )(keys_col, keys_row)
    return ranks3d.reshape(b, n)


def _make_sc_kernel(nb, n, c, k):
    info = plsc.get_sparse_core_info()
    nc, ns = info.num_cores, info.num_subcores
    nw = nc * ns
    rows_w = k // nw
    rchunk = 8
    nch = rows_w // rchunk
    mesh = plsc.VectorSubcoreMesh(core_axis_name="c", subcore_axis_name="s")

    @functools.partial(
        pl.kernel,
        mesh=mesh,
        compiler_params=pltpu.CompilerParams(needs_layout_passes=False),
        out_type=(
            jax.ShapeDtypeStruct((nb * k,), jnp.float32),
            jax.ShapeDtypeStruct((nb * k,), jnp.int32),
            jax.ShapeDtypeStruct((nb * k, c), jnp.float32),
            jax.ShapeDtypeStruct((nb * k * k,), jnp.float32),
        ),
        scratch_types=[
            pltpu.VMEM((n,), jnp.float32),
            pltpu.VMEM((n,), jnp.int32),
            pltpu.VMEM((k,), jnp.float32),
            pltpu.VMEM((k,), jnp.int32),
            pltpu.VMEM((k,), jnp.int32),
            pltpu.VMEM((rows_w,), jnp.int32),
            pltpu.VMEM((rows_w, c), jnp.float32),
            pltpu.VMEM((rchunk, n), jnp.float32),
            pltpu.VMEM((rchunk, n), jnp.float32),
            pltpu.VMEM((rchunk * k,), jnp.float32),
            pltpu.VMEM((rchunk * k,), jnp.float32),
            pltpu.VMEM((n,), jnp.int32),
            pltpu.VMEM((n,), jnp.int32),
            pltpu.VMEM((3 * _LANES,), jnp.int32),
            pltpu.VMEM_SHARED((nb * k,), jnp.int32),
            pltpu.SemaphoreType.DMA,
            pltpu.SemaphoreType.DMA,
            pltpu.SemaphoreType.DMA,
            pltpu.SemaphoreType.DMA,
            pltpu.SemaphoreType.DMA,
        ],
    )
    def sc_kernel(scores_hbm, ranks_hbm, feat_hbm, adj_hbm,
                  tkv_hbm, tki_hbm, selfeat_hbm, seladj_hbm,
                  scores_v, ranks_v, scatv_v, scati_v, idxfull_v,
                  fidx_v, feat_v, rows0, rows1, orow0, orow1, cnt_buf,
                  tmp_v, shift_v, idx_sh,
                  sem, insem0, insem1, outsem0, outsem1):
        cid = lax.axis_index("c")
        sid = lax.axis_index("s")
        wid = sid * nc + cid
        insems = (insem0, insem1)
        outsems = (outsem0, outsem1)
        rowbufs = (rows0, rows1)
        obufs = (orow0, orow1)

        @pl.when(sid < nb)
        def _():
            bb = sid
            pltpu.sync_copy(scores_hbm.at[pl.ds(bb * n, n)], scores_v)
            pltpu.sync_copy(ranks_hbm.at[pl.ds(bb * n, n)], ranks_v)

            def zero_body(t, carry):
                cnt_buf[pl.ds(t * _LANES, _LANES)] = jnp.zeros(
                    (_LANES,), jnp.int32
                )
                return carry

            lax.fori_loop(0, n // _LANES, zero_body, jnp.int32(0))
            sent = jnp.full((_LANES,), -1, jnp.int32)
            shift_v[pl.ds(0, _LANES)] = sent
            shift_v[pl.ds(2 * _LANES, _LANES)] = sent
            lid = lax.iota(jnp.int32, _LANES)

            def scat_body(t, carry):
                sv = scores_v[pl.ds(t * _LANES, _LANES)]
                rv = ranks_v[pl.ds(t * _LANES, _LANES)]
                gi = t * _LANES + lid
                prior = plsc.load_gather(cnt_buf, [rv])
                plsc.store_scatter(tmp_v, [rv], lid)
                got = plsc.load_gather(tmp_v, [rv])
                ndup = jnp.sum((got != lid).astype(jnp.int32))

                @pl.when(ndup == 0)
                def _():
                    p = rv + prior
                    msk = p < k
                    plsc.store_scatter(scatv_v, [p], sv, mask=msk)
                    plsc.store_scatter(scati_v, [p], gi, mask=msk)
                    plsc.store_scatter(cnt_buf, [rv], prior + 1)

                @pl.when(ndup != 0)
                def _():
                    shift_v[pl.ds(_LANES, _LANES)] = rv
                    wcnt = jnp.zeros((_LANES,), jnp.int32)
                    back = jnp.zeros((_LANES,), jnp.int32)
                    for sh in range(1, _LANES):
                        f = plsc.load_gather(shift_v, [lid + (_LANES - sh)])
                        wcnt += (f == rv).astype(jnp.int32)
                        bwd = plsc.load_gather(shift_v, [lid + (_LANES + sh)])
                        back += (bwd == rv).astype(jnp.int32)
                    p = rv + prior + wcnt
                    msk = p < k
                    plsc.store_scatter(scatv_v, [p], sv, mask=msk)
                    plsc.store_scatter(scati_v, [p], gi, mask=msk)
                    plsc.store_scatter(
                        cnt_buf, [rv], prior + wcnt + 1, mask=back == 0
                    )

                return carry

            lax.fori_loop(0, n // _LANES, scat_body, jnp.int32(0))
            pltpu.sync_copy(scati_v, idx_sh.at[pl.ds(bb * k, k)])

            @pl.when(cid == 0)
            def _():
                pltpu.sync_copy(scatv_v, tkv_hbm.at[pl.ds(bb * k, k)])
                pltpu.sync_copy(scati_v, tki_hbm.at[pl.ds(bb * k, k)])

        plsc.subcore_barrier()

        base = wid * rows_w

        def start_in(ch, slot):
            pltpu.async_copy(
                adj_hbm.at[idxfull_v.at[pl.ds(base + ch * rchunk, rchunk)]],
                rowbufs[slot],
                insems[slot],
            )

        def batch_body(bb, carry):
            pltpu.sync_copy(idx_sh.at[pl.ds(bb * k, k)], idxfull_v)

            def fidx_body(t, cy):
                fidx_v[pl.ds(t * _LANES, _LANES)] = (
                    idxfull_v[pl.ds(base + t * _LANES, _LANES)] + bb * n
                )
                return cy

            lax.fori_loop(0, rows_w // _LANES, fidx_body, jnp.int32(0))
            pltpu.async_copy(feat_hbm.at[fidx_v], feat_v, sem).wait()
            pltpu.sync_copy(feat_v, selfeat_hbm.at[pl.ds(bb * k + base, rows_w)])

            start_in(0, 0)
            for ch in range(nch):
                s = ch % 2
                rbuf = rowbufs[s]
                obuf = obufs[s]
                pltpu.make_async_copy(
                    adj_hbm.at[idxfull_v.at[pl.ds(base, rchunk)]], rbuf, insems[s]
                ).wait()
                if ch + 1 < nch:
                    start_in(ch + 1, 1 - s)
                if ch >= 2:
                    pltpu.make_async_copy(
                        obuf, seladj_hbm.at[pl.ds(0, rchunk * k)], outsems[s]
                    ).wait()
                else:
                    @pl.when(bb > 0)
                    def _():
                        pltpu.make_async_copy(
                            obuf, seladj_hbm.at[pl.ds(0, rchunk * k)], outsems[s]
                        ).wait()

                @plsc.parallel_loop(0, k // _LANES, unroll=2)
                def _(g):
                    cols = idxfull_v[pl.ds(g * _LANES, _LANES)]
                    for r in range(rchunk):
                        rsel = jnp.full((_LANES,), r, jnp.int32)
                        obuf[pl.ds(r * k + g * _LANES, _LANES)] = plsc.load_gather(
                            rbuf, [rsel, cols]
                        )

                rbase = base + ch * rchunk
                pltpu.async_copy(
                    obuf,
                    seladj_hbm.at[pl.ds((bb * k + rbase) * k, rchunk * k)],
                    outsems[s],
                )
            return carry

        lax.fori_loop(0, nb, batch_body, jnp.int32(0))
        pltpu.make_async_copy(
            orow0, seladj_hbm.at[pl.ds(0, rchunk * k)], outsem0
        ).wait()
        pltpu.make_async_copy(
            orow1, seladj_hbm.at[pl.ds(0, rchunk * k)], outsem1
        ).wait()

    return sc_kernel


def kernel(importance_scores, features, adjacency_matrix):
    b, n, c = features.shape
    k = max(1, int(n * 0.5))
    ranks = _compute_ranks(importance_scores)
    cp = 128
    feat_flat = jnp.pad(features.reshape(b * n, c), ((0, 0), (0, cp - c)))
    sc = _make_sc_kernel(b, n, cp, k)
    tkv, tki, selfeat_flat, seladj_flat = sc(
        importance_scores.reshape(-1), ranks.reshape(-1), feat_flat,
        adjacency_matrix
    )
    return (
        selfeat_flat.reshape(b, k, cp)[:, :, :c],
        tki.reshape(b, k),
        seladj_flat.reshape(b, k, k),
        tkv.reshape(b, k),
    )


# speedup vs baseline: 1.0312x; 1.0312x over previous
import functools

import jax
import jax.numpy as jnp
from jax import lax
from jax.experimental import pallas as pl
from jax.experimental.pallas import tpu as pltpu
from jax.experimental.pallas import tpu_sc as plsc


_LANES = 16


def _sort_key(x):
    xz = x + 0.0
    b = lax.bitcast_convert_type(xz, jnp.int32)
    neg = jnp.bitwise_xor(jnp.bitwise_not(b), jnp.int32(-2147483648))
    return jnp.where(b >= 0, b, neg)


def _keys_kernel(s_ref, k_ref):
    k_ref[...] = _sort_key(s_ref[...])


def _count_kernel(kcol_ref, krow_ref, out_ref, acc_ref, *, bj, njt):
    jt = pl.program_id(2)
    m = (krow_ref[0] > kcol_ref[0]).astype(jnp.int32)

    @pl.when(jt == 0)
    def _():
        acc_ref[...] = m

    @pl.when(jt != 0)
    def _():
        acc_ref[...] += m

    @pl.when(jt == njt - 1)
    def _():
        out_ref[0] = jnp.sum(acc_ref[...], axis=1, keepdims=True)


def _compute_ranks(scores):
    b, n = scores.shape
    keys = pl.pallas_call(
        _keys_kernel,
        out_shape=jax.ShapeDtypeStruct((b, n), jnp.int32),
    )(scores)
    keys_col = keys.reshape(b, n, 1)
    keys_row = keys.reshape(b, 1, n)

    bi, bj = 2048, 1024
    nit, njt = n // bi, n // bj

    ranks3d = pl.pallas_call(
        functools.partial(_count_kernel, bj=bj, njt=njt),
        grid=(b, nit, njt),
        in_specs=[
            pl.BlockSpec((1, bi, 1), lambda bb, it, jt: (bb, it, 0)),
            pl.BlockSpec((1, 1, bj), lambda bb, it, jt: (bb, 0, jt)),
        ],
        out_specs=pl.BlockSpec((1, bi, 1), lambda bb, it, jt: (bb, it, 0)),
        out_shape=jax.ShapeDtypeStruct((b, n, 1), jnp.int32),
        scratch_shapes=[pltpu.VMEM((bi, bj), jnp.int32)],
    )(keys_col, keys_row)
    return ranks3d.reshape(b, n)


def _make_sc_kernel(nb, n, c, k):
    info = plsc.get_sparse_core_info()
    nc, ns = info.num_cores, info.num_subcores
    nw = nc * ns
    rows_w = k // nw
    rchunk = 8
    nch = rows_w // rchunk
    mesh = plsc.VectorSubcoreMesh(core_axis_name="c", subcore_axis_name="s")

    @functools.partial(
        pl.kernel,
        mesh=mesh,
        compiler_params=pltpu.CompilerParams(needs_layout_passes=False),
        out_type=(
            jax.ShapeDtypeStruct((nb * k,), jnp.float32),
            jax.ShapeDtypeStruct((nb * k,), jnp.int32),
            jax.ShapeDtypeStruct((nb * k, c), jnp.float32),
            jax.ShapeDtypeStruct((nb * k * k,), jnp.float32),
        ),
        scratch_types=[
            pltpu.VMEM((n,), jnp.float32),
            pltpu.VMEM((n,), jnp.int32),
            pltpu.VMEM((k,), jnp.float32),
            pltpu.VMEM((k,), jnp.int32),
            pltpu.VMEM((k,), jnp.int32),
            pltpu.VMEM((rows_w,), jnp.int32),
            pltpu.VMEM((rows_w, c), jnp.float32),
            pltpu.VMEM((rchunk, n), jnp.float32),
            pltpu.VMEM((rchunk, n), jnp.float32),
            pltpu.VMEM((rchunk * k,), jnp.float32),
            pltpu.VMEM((rchunk * k,), jnp.float32),
            pltpu.VMEM((n,), jnp.int32),
            pltpu.VMEM((n,), jnp.int32),
            pltpu.VMEM((3 * _LANES,), jnp.int32),
            pltpu.VMEM_SHARED((nb * k,), jnp.int32),
            pltpu.SemaphoreType.DMA,
            pltpu.SemaphoreType.DMA,
            pltpu.SemaphoreType.DMA,
            pltpu.SemaphoreType.DMA,
            pltpu.SemaphoreType.DMA,
        ],
    )
    def sc_kernel(scores_hbm, ranks_hbm, feat_hbm, adj_hbm,
                  tkv_hbm, tki_hbm, selfeat_hbm, seladj_hbm,
                  scores_v, ranks_v, scatv_v, scati_v, idxfull_v,
                  fidx_v, feat_v, rows0, rows1, orow0, orow1, cnt_buf,
                  tmp_v, shift_v, idx_sh,
                  sem, insem0, insem1, outsem0, outsem1):
        cid = lax.axis_index("c")
        sid = lax.axis_index("s")
        wid = sid * nc + cid
        insems = (insem0, insem1)
        outsems = (outsem0, outsem1)
        rowbufs = (rows0, rows1)
        obufs = (orow0, orow1)

        @pl.when(sid < nb)
        def _():
            bb = sid
            pltpu.sync_copy(scores_hbm.at[pl.ds(bb * n, n)], scores_v)
            pltpu.sync_copy(ranks_hbm.at[pl.ds(bb * n, n)], ranks_v)

            def zero_body(t, carry):
                cnt_buf[pl.ds(t * _LANES, _LANES)] = jnp.zeros(
                    (_LANES,), jnp.int32
                )
                return carry

            lax.fori_loop(0, n // _LANES, zero_body, jnp.int32(0))
            sent = jnp.full((_LANES,), -1, jnp.int32)
            shift_v[pl.ds(0, _LANES)] = sent
            shift_v[pl.ds(2 * _LANES, _LANES)] = sent
            lid = lax.iota(jnp.int32, _LANES)

            def scat_body(t, carry):
                sv = scores_v[pl.ds(t * _LANES, _LANES)]
                rv = ranks_v[pl.ds(t * _LANES, _LANES)]
                gi = t * _LANES + lid
                prior = plsc.load_gather(cnt_buf, [rv])
                plsc.store_scatter(tmp_v, [rv], lid)
                got = plsc.load_gather(tmp_v, [rv])
                ndup = jnp.sum((got != lid).astype(jnp.int32))

                @pl.when(ndup == 0)
                def _():
                    p = rv + prior
                    msk = p < k
                    plsc.store_scatter(scatv_v, [p], sv, mask=msk)
                    plsc.store_scatter(scati_v, [p], gi, mask=msk)
                    plsc.store_scatter(cnt_buf, [rv], prior + 1)

                @pl.when(ndup != 0)
                def _():
                    shift_v[pl.ds(_LANES, _LANES)] = rv
                    wcnt = jnp.zeros((_LANES,), jnp.int32)
                    back = jnp.zeros((_LANES,), jnp.int32)
                    for sh in range(1, _LANES):
                        f = plsc.load_gather(shift_v, [lid + (_LANES - sh)])
                        wcnt += (f == rv).astype(jnp.int32)
                        bwd = plsc.load_gather(shift_v, [lid + (_LANES + sh)])
                        back += (bwd == rv).astype(jnp.int32)
                    p = rv + prior + wcnt
                    msk = p < k
                    plsc.store_scatter(scatv_v, [p], sv, mask=msk)
                    plsc.store_scatter(scati_v, [p], gi, mask=msk)
                    plsc.store_scatter(
                        cnt_buf, [rv], prior + wcnt + 1, mask=back == 0
                    )

                return carry

            lax.fori_loop(0, n // _LANES, scat_body, jnp.int32(0))
            pltpu.sync_copy(scati_v, idx_sh.at[pl.ds(bb * k, k)])

            @pl.when(cid == 0)
            def _():
                pltpu.sync_copy(scatv_v, tkv_hbm.at[pl.ds(bb * k, k)])
                pltpu.sync_copy(scati_v, tki_hbm.at[pl.ds(bb * k, k)])

        plsc.subcore_barrier()

        base = wid * rows_w

        def start_in(ch, slot):
            pltpu.async_copy(
                adj_hbm.at[idxfull_v.at[pl.ds(base + ch * rchunk, rchunk)]],
                rowbufs[slot],
                insems[slot],
            )

        def batch_body(bb, carry):
            pltpu.sync_copy(idx_sh.at[pl.ds(bb * k, k)], idxfull_v)

            def fidx_body(t, cy):
                fidx_v[pl.ds(t * _LANES, _LANES)] = (
                    idxfull_v[pl.ds(base + t * _LANES, _LANES)] + bb * n
                )
                return cy

            lax.fori_loop(0, rows_w // _LANES, fidx_body, jnp.int32(0))
            pltpu.async_copy(feat_hbm.at[fidx_v], feat_v, sem).wait()
            pltpu.sync_copy(feat_v, selfeat_hbm.at[pl.ds(bb * k + base, rows_w)])

            start_in(0, 0)
            for ch in range(nch):
                s = ch % 2
                rbuf = rowbufs[s]
                obuf = obufs[s]
                pltpu.make_async_copy(
                    adj_hbm.at[idxfull_v.at[pl.ds(base, rchunk)]], rbuf, insems[s]
                ).wait()
                if ch + 1 < nch:
                    start_in(ch + 1, 1 - s)
                if ch >= 2:
                    pltpu.make_async_copy(
                        obuf, seladj_hbm.at[pl.ds(0, rchunk * k)], outsems[s]
                    ).wait()
                else:
                    @pl.when(bb > 0)
                    def _():
                        pltpu.make_async_copy(
                            obuf, seladj_hbm.at[pl.ds(0, rchunk * k)], outsems[s]
                        ).wait()

                @plsc.parallel_loop(0, k // _LANES, unroll=2)
                def _(g):
                    cols = idxfull_v[pl.ds(g * _LANES, _LANES)]
                    for r in range(rchunk):
                        rsel = jnp.full((_LANES,), r, jnp.int32)
                        obuf[pl.ds(r * k + g * _LANES, _LANES)] = plsc.load_gather(
                            rbuf, [rsel, cols]
                        )

                rbase = base + ch * rchunk
                pltpu.async_copy(
                    obuf,
                    seladj_hbm.at[pl.ds((bb * k + rbase) * k, rchunk * k)],
                    outsems[s],
                )
            return carry

        lax.fori_loop(0, nb, batch_body, jnp.int32(0))
        pltpu.make_async_copy(
            orow0, seladj_hbm.at[pl.ds(0, rchunk * k)], outsem0
        ).wait()
        pltpu.make_async_copy(
            orow1, seladj_hbm.at[pl.ds(0, rchunk * k)], outsem1
        ).wait()

    return sc_kernel


def kernel(importance_scores, features, adjacency_matrix):
    b, n, c = features.shape
    k = max(1, int(n * 0.5))
    ranks = _compute_ranks(importance_scores)
    cp = 128
    feat_flat = jnp.pad(features.reshape(b * n, c), ((0, 0), (0, cp - c)))
    sc = _make_sc_kernel(b, n, cp, k)
    tkv, tki, selfeat_flat, seladj_flat = sc(
        importance_scores.reshape(-1), ranks.reshape(-1), feat_flat,
        adjacency_matrix
    )
    return (
        selfeat_flat.reshape(b, k, cp)[:, :, :c],
        tki.reshape(b, k),
        seladj_flat.reshape(b, k, k),
        tkv.reshape(b, k),
    )


# speedup vs baseline: 1.0411x; 1.0096x over previous
import functools

import jax
import jax.numpy as jnp
from jax import lax
from jax.experimental import pallas as pl
from jax.experimental.pallas import tpu as pltpu
from jax.experimental.pallas import tpu_sc as plsc


_LANES = 16


def _sort_key(x):
    xz = x + 0.0
    b = lax.bitcast_convert_type(xz, jnp.int32)
    neg = jnp.bitwise_xor(jnp.bitwise_not(b), jnp.int32(-2147483648))
    return jnp.where(b >= 0, b, neg)


def _keys_kernel(s_ref, k_ref):
    k_ref[...] = _sort_key(s_ref[...])


def _count_kernel(kcol_ref, krow_ref, out_ref, acc_ref, *, bj, njt):
    jt = pl.program_id(2)
    m = (krow_ref[0] > kcol_ref[0]).astype(jnp.int32)

    @pl.when(jt == 0)
    def _():
        acc_ref[...] = m

    @pl.when(jt != 0)
    def _():
        acc_ref[...] += m

    @pl.when(jt == njt - 1)
    def _():
        out_ref[0] = jnp.sum(acc_ref[...], axis=1, keepdims=True)


def _compute_ranks(scores):
    b, n = scores.shape
    keys = pl.pallas_call(
        _keys_kernel,
        out_shape=jax.ShapeDtypeStruct((b, n), jnp.int32),
    )(scores)
    keys_col = keys.reshape(b, n, 1)
    keys_row = keys.reshape(b, 1, n)

    bi, bj = 4096, 1024
    nit, njt = n // bi, n // bj

    ranks3d = pl.pallas_call(
        functools.partial(_count_kernel, bj=bj, njt=njt),
        grid=(b, nit, njt),
        in_specs=[
            pl.BlockSpec((1, bi, 1), lambda bb, it, jt: (bb, it, 0)),
            pl.BlockSpec((1, 1, bj), lambda bb, it, jt: (bb, 0, jt)),
        ],
        out_specs=pl.BlockSpec((1, bi, 1), lambda bb, it, jt: (bb, it, 0)),
        out_shape=jax.ShapeDtypeStruct((b, n, 1), jnp.int32),
        scratch_shapes=[pltpu.VMEM((bi, bj), jnp.int32)],
    )(keys_col, keys_row)
    return ranks3d.reshape(b, n)


def _make_sc_kernel(nb, n, c, k):
    info = plsc.get_sparse_core_info()
    nc, ns = info.num_cores, info.num_subcores
    nw = nc * ns
    rows_w = k // nw
    rchunk = 8
    nch = rows_w // rchunk
    mesh = plsc.VectorSubcoreMesh(core_axis_name="c", subcore_axis_name="s")

    @functools.partial(
        pl.kernel,
        mesh=mesh,
        compiler_params=pltpu.CompilerParams(needs_layout_passes=False),
        out_type=(
            jax.ShapeDtypeStruct((nb * k,), jnp.float32),
            jax.ShapeDtypeStruct((nb * k,), jnp.int32),
            jax.ShapeDtypeStruct((nb * k, c), jnp.float32),
            jax.ShapeDtypeStruct((nb * k * k,), jnp.float32),
        ),
        scratch_types=[
            pltpu.VMEM((n,), jnp.float32),
            pltpu.VMEM((n,), jnp.int32),
            pltpu.VMEM((k,), jnp.float32),
            pltpu.VMEM((k,), jnp.int32),
            pltpu.VMEM((k,), jnp.int32),
            pltpu.VMEM((rows_w,), jnp.int32),
            pltpu.VMEM((rows_w, c), jnp.float32),
            pltpu.VMEM((rchunk, n), jnp.float32),
            pltpu.VMEM((rchunk, n), jnp.float32),
            pltpu.VMEM((rchunk * k,), jnp.float32),
            pltpu.VMEM((rchunk * k,), jnp.float32),
            pltpu.VMEM((n,), jnp.int32),
            pltpu.VMEM((n,), jnp.int32),
            pltpu.VMEM((3 * _LANES,), jnp.int32),
            pltpu.VMEM_SHARED((nb * k,), jnp.int32),
            pltpu.SemaphoreType.DMA,
            pltpu.SemaphoreType.DMA,
            pltpu.SemaphoreType.DMA,
            pltpu.SemaphoreType.DMA,
            pltpu.SemaphoreType.DMA,
        ],
    )
    def sc_kernel(scores_hbm, ranks_hbm, feat_hbm, adj_hbm,
                  tkv_hbm, tki_hbm, selfeat_hbm, seladj_hbm,
                  scores_v, ranks_v, scatv_v, scati_v, idxfull_v,
                  fidx_v, feat_v, rows0, rows1, orow0, orow1, cnt_buf,
                  tmp_v, shift_v, idx_sh,
                  sem, insem0, insem1, outsem0, outsem1):
        cid = lax.axis_index("c")
        sid = lax.axis_index("s")
        wid = sid * nc + cid
        insems = (insem0, insem1)
        outsems = (outsem0, outsem1)
        rowbufs = (rows0, rows1)
        obufs = (orow0, orow1)

        @pl.when(sid < nb)
        def _():
            bb = sid
            pltpu.sync_copy(scores_hbm.at[pl.ds(bb * n, n)], scores_v)
            pltpu.sync_copy(ranks_hbm.at[pl.ds(bb * n, n)], ranks_v)

            def zero_body(t, carry):
                cnt_buf[pl.ds(t * _LANES, _LANES)] = jnp.zeros(
                    (_LANES,), jnp.int32
                )
                return carry

            lax.fori_loop(0, n // _LANES, zero_body, jnp.int32(0))
            sent = jnp.full((_LANES,), -1, jnp.int32)
            shift_v[pl.ds(0, _LANES)] = sent
            shift_v[pl.ds(2 * _LANES, _LANES)] = sent
            lid = lax.iota(jnp.int32, _LANES)

            def scat_body(t, carry):
                sv = scores_v[pl.ds(t * _LANES, _LANES)]
                rv = ranks_v[pl.ds(t * _LANES, _LANES)]
                gi = t * _LANES + lid
                prior = plsc.load_gather(cnt_buf, [rv])
                plsc.store_scatter(tmp_v, [rv], lid)
                got = plsc.load_gather(tmp_v, [rv])
                ndup = jnp.sum((got != lid).astype(jnp.int32))

                @pl.when(ndup == 0)
                def _():
                    p = rv + prior
                    msk = p < k
                    plsc.store_scatter(scatv_v, [p], sv, mask=msk)
                    plsc.store_scatter(scati_v, [p], gi, mask=msk)
                    plsc.store_scatter(cnt_buf, [rv], prior + 1)

                @pl.when(ndup != 0)
                def _():
                    shift_v[pl.ds(_LANES, _LANES)] = rv
                    wcnt = jnp.zeros((_LANES,), jnp.int32)
                    back = jnp.zeros((_LANES,), jnp.int32)
                    for sh in range(1, _LANES):
                        f = plsc.load_gather(shift_v, [lid + (_LANES - sh)])
                        wcnt += (f == rv).astype(jnp.int32)
                        bwd = plsc.load_gather(shift_v, [lid + (_LANES + sh)])
                        back += (bwd == rv).astype(jnp.int32)
                    p = rv + prior + wcnt
                    msk = p < k
                    plsc.store_scatter(scatv_v, [p], sv, mask=msk)
                    plsc.store_scatter(scati_v, [p], gi, mask=msk)
                    plsc.store_scatter(
                        cnt_buf, [rv], prior + wcnt + 1, mask=back == 0
                    )

                return carry

            lax.fori_loop(0, n // _LANES, scat_body, jnp.int32(0))
            pltpu.sync_copy(scati_v, idx_sh.at[pl.ds(bb * k, k)])

            @pl.when(cid == 0)
            def _():
                pltpu.sync_copy(scatv_v, tkv_hbm.at[pl.ds(bb * k, k)])
                pltpu.sync_copy(scati_v, tki_hbm.at[pl.ds(bb * k, k)])

        plsc.subcore_barrier()

        base = wid * rows_w

        def start_in(ch, slot):
            pltpu.async_copy(
                adj_hbm.at[idxfull_v.at[pl.ds(base + ch * rchunk, rchunk)]],
                rowbufs[slot],
                insems[slot],
            )

        def batch_body(bb, carry):
            pltpu.sync_copy(idx_sh.at[pl.ds(bb * k, k)], idxfull_v)

            def fidx_body(t, cy):
                fidx_v[pl.ds(t * _LANES, _LANES)] = (
                    idxfull_v[pl.ds(base + t * _LANES, _LANES)] + bb * n
                )
                return cy

            lax.fori_loop(0, rows_w // _LANES, fidx_body, jnp.int32(0))
            pltpu.async_copy(feat_hbm.at[fidx_v], feat_v, sem).wait()
            pltpu.sync_copy(feat_v, selfeat_hbm.at[pl.ds(bb * k + base, rows_w)])

            start_in(0, 0)
            for ch in range(nch):
                s = ch % 2
                rbuf = rowbufs[s]
                obuf = obufs[s]
                pltpu.make_async_copy(
                    adj_hbm.at[idxfull_v.at[pl.ds(base, rchunk)]], rbuf, insems[s]
                ).wait()
                if ch + 1 < nch:
                    start_in(ch + 1, 1 - s)
                if ch >= 2:
                    pltpu.make_async_copy(
                        obuf, seladj_hbm.at[pl.ds(0, rchunk * k)], outsems[s]
                    ).wait()
                else:
                    @pl.when(bb > 0)
                    def _():
                        pltpu.make_async_copy(
                            obuf, seladj_hbm.at[pl.ds(0, rchunk * k)], outsems[s]
                        ).wait()

                @plsc.parallel_loop(0, k // _LANES, unroll=2)
                def _(g):
                    cols = idxfull_v[pl.ds(g * _LANES, _LANES)]
                    for r in range(rchunk):
                        rsel = jnp.full((_LANES,), r, jnp.int32)
                        obuf[pl.ds(r * k + g * _LANES, _LANES)] = plsc.load_gather(
                            rbuf, [rsel, cols]
                        )

                rbase = base + ch * rchunk
                pltpu.async_copy(
                    obuf,
                    seladj_hbm.at[pl.ds((bb * k + rbase) * k, rchunk * k)],
                    outsems[s],
                )
            return carry

        lax.fori_loop(0, nb, batch_body, jnp.int32(0))
        pltpu.make_async_copy(
            orow0, seladj_hbm.at[pl.ds(0, rchunk * k)], outsem0
        ).wait()
        pltpu.make_async_copy(
            orow1, seladj_hbm.at[pl.ds(0, rchunk * k)], outsem1
        ).wait()

    return sc_kernel


def kernel(importance_scores, features, adjacency_matrix):
    b, n, c = features.shape
    k = max(1, int(n * 0.5))
    ranks = _compute_ranks(importance_scores)
    cp = 128
    feat_flat = jnp.pad(features.reshape(b * n, c), ((0, 0), (0, cp - c)))
    sc = _make_sc_kernel(b, n, cp, k)
    tkv, tki, selfeat_flat, seladj_flat = sc(
        importance_scores.reshape(-1), ranks.reshape(-1), feat_flat,
        adjacency_matrix
    )
    return (
        selfeat_flat.reshape(b, k, cp)[:, :, :c],
        tki.reshape(b, k),
        seladj_flat.reshape(b, k, k),
        tkv.reshape(b, k),
    )
